# baseline (speedup 1.0000x reference)
import functools
import jax
import jax.numpy as jnp
from jax import lax
from jax.experimental import pallas as pl
from jax.experimental.pallas import tpu as pltpu
from jax.experimental.pallas import tpu_sc as plsc

N = 10000
E = 320000
D = 128
H = 128
DE = 16
GD = 64
A = 64
B = 8

NC = 2
NS = 16
NW = NC * NS
EW = E // NW
C = 40
NCHUNK = EW // C
NB = 5
NLOOP = NCHUNK // NB
RPT = 640
RLAST = N - (NS - 1) * RPT

_f32 = jnp.float32


def _mm1_body(x_ref, wm_ref, ws_ref, b_ref, hw_ref, hs_ref):
    xb = x_ref[...]
    hw_ref[...] = jnp.dot(xb, wm_ref[...], preferred_element_type=_f32)
    hs_ref[...] = jnp.dot(xb, ws_ref[...], preferred_element_type=_f32) + b_ref[...]


def _mm1(x, wm, ws, b, bn=2000):
    din = x.shape[1]
    grid = (N // bn,)
    return pl.pallas_call(
        _mm1_body,
        grid=grid,
        in_specs=[
            pl.BlockSpec((bn, din), lambda i: (i, 0)),
            pl.BlockSpec((din, H), lambda i: (0, 0)),
            pl.BlockSpec((din, H), lambda i: (0, 0)),
            pl.BlockSpec((1, H), lambda i: (0, 0)),
        ],
        out_specs=[
            pl.BlockSpec((bn, H), lambda i: (i, 0)),
            pl.BlockSpec((bn, H), lambda i: (i, 0)),
        ],
        out_shape=[
            jax.ShapeDtypeStruct((N, H), _f32),
            jax.ShapeDtypeStruct((N, H), _f32),
        ],
    )(x, wm, ws, b.reshape(1, H))


def _mm2_body(hs_ref, p_ref, wm_ref, ws_ref, b_ref, hw_ref, hs_o_ref):
    h = jnp.maximum(hs_ref[...] + p_ref[0] + p_ref[1], 0.0)
    hw_ref[...] = jnp.dot(h, wm_ref[...], preferred_element_type=_f32)
    hs_o_ref[...] = jnp.dot(h, ws_ref[...], preferred_element_type=_f32) + b_ref[...]


def _mm2(hs, p, wm, ws, b, bn=2000):
    grid = (N // bn,)
    return pl.pallas_call(
        _mm2_body,
        grid=grid,
        in_specs=[
            pl.BlockSpec((bn, H), lambda i: (i, 0)),
            pl.BlockSpec((2, bn, H), lambda i: (0, i, 0)),
            pl.BlockSpec((H, H), lambda i: (0, 0)),
            pl.BlockSpec((H, H), lambda i: (0, 0)),
            pl.BlockSpec((1, H), lambda i: (0, 0)),
        ],
        out_specs=[
            pl.BlockSpec((bn, H), lambda i: (i, 0)),
            pl.BlockSpec((bn, H), lambda i: (i, 0)),
        ],
        out_shape=[
            jax.ShapeDtypeStruct((N, H), _f32),
            jax.ShapeDtypeStruct((N, H), _f32),
        ],
    )(hs, p, wm, ws, b.reshape(1, H))


def _mme_body(ea_ref, we_ref, o_ref):
    o = jnp.dot(ea_ref[...], we_ref[...], preferred_element_type=_f32)
    bits = jax.lax.bitcast_convert_type(
        o.astype(jnp.bfloat16).astype(_f32), jnp.int32)
    pieces = []
    for g in range(4):
        lo = jax.lax.shift_right_logical(bits[:, 32 * g:32 * g + 16], 16)
        hi = bits[:, 32 * g + 16:32 * g + 32] & jnp.int32(-65536)
        pieces.append(lo | hi)
    o_ref[...] = jnp.concatenate(pieces, axis=1)


def _mme(edge_attr, we, be=8000):
    grid = (E // be,)
    return pl.pallas_call(
        _mme_body,
        grid=grid,
        in_specs=[
            pl.BlockSpec((be, DE), lambda i: (i, 0)),
            pl.BlockSpec((DE, H), lambda i: (0, 0)),
        ],
        out_specs=pl.BlockSpec((be, H // 2), lambda i: (i, 0)),
        out_shape=jax.ShapeDtypeStruct((E, H // 2), jnp.int32),
    )(edge_attr, we)


def _sc_body(hw_hbm, ew_hbm, src_hbm, dst_hbm, zrs_hbm, out_hbm,
             sidx, didx, ewbf, hwb, agg, *sems):
    sem_si = sems[0 * NB:1 * NB]
    sem_di = sems[1 * NB:2 * NB]
    sem_ew = sems[2 * NB:3 * NB]
    sem_hw = sems[3 * NB:4 * NB]
    sem_sc = sems[4 * NB:5 * NB]
    c = lax.axis_index("c")
    s = lax.axis_index("s")
    wid = s * NC + c
    ebase = wid * EW

    rbase = pl.multiple_of(s * RPT, 8)

    @pl.when(s < NS - 1)
    def _():
        pltpu.sync_copy(zrs_hbm, agg.at[pl.ds(rbase, RPT)])

    @pl.when(s == NS - 1)
    def _():
        pltpu.sync_copy(zrs_hbm.at[pl.ds(0, RLAST)],
                        agg.at[pl.ds((NS - 1) * RPT, RLAST)])

    plsc.subcore_barrier()

    def load(j, b):
        off = pl.multiple_of(ebase + j * C, 8)
        pltpu.async_copy(src_hbm.at[pl.ds(off, C)], sidx.at[b], sem_si[b])
        pltpu.async_copy(dst_hbm.at[pl.ds(off, C)], didx.at[b], sem_di[b])
        pltpu.async_copy(ew_hbm.at[wid * NCHUNK + j], ewbf.at[b], sem_ew[b])

    def gath(b):
        pltpu.make_async_copy(src_hbm.at[pl.ds(0, C)], sidx.at[b],
                              sem_si[b]).wait()
        pltpu.async_copy(hw_hbm.at[sidx.at[b]], hwb.at[b], sem_hw[b])

    def scat(b):
        pltpu.make_async_copy(hw_hbm.at[sidx.at[b]], hwb.at[b],
                              sem_hw[b]).wait()
        pltpu.make_async_copy(ew_hbm.at[0], ewbf.at[b], sem_ew[b]).wait()
        pltpu.make_async_copy(dst_hbm.at[pl.ds(0, C)], didx.at[b],
                              sem_di[b]).wait()

        mask = jnp.int32(-65536)

        def row(rr, rc):
            for half in range(2):
                r = 2 * rr + half
                for g in range(4):
                    e = ewbf[b, rr, pl.ds(half * 64 + g * 16, 16)]
                    ea = jax.lax.bitcast_convert_type(
                        jax.lax.shift_left(e, 16), _f32)
                    eb = jax.lax.bitcast_convert_type(e & mask, _f32)
                    sa = pl.ds(g * 32, 16)
                    sb = pl.ds(g * 32 + 16, 16)
                    hwb[b, r, sa] = jnp.maximum(hwb[b, r, sa] + ea, 0.0)
                    hwb[b, r, sb] = jnp.maximum(hwb[b, r, sb] + eb, 0.0)
            return rc

        lax.fori_loop(0, C // 2, row, 0)
        pltpu.async_copy(hwb.at[b], agg.at[didx.at[b]], sem_sc[b], add=True)

    def wait_scat(b):
        pltpu.make_async_copy(hwb.at[b], agg.at[didx.at[b]],
                              sem_sc[b]).wait()

    def body(t, first, last):
        for b in range(NB):
            j = t * NB + b
            bl = (b + 3) % NB
            if first and b < 2:
                pass
            else:
                wait_scat(bl)
            if not (last and b >= 2):
                load(j + 3, bl)
            if not (last and b == NB - 1):
                gath((b + 1) % NB)
            scat(b)

    for j in range(3):
        load(j, j)
    gath(0)

    body(0, True, False)
    lax.fori_loop(1, NLOOP - 1, lambda t, cy: (body(t, False, False), cy)[1],
                  0)
    body(NLOOP - 1, False, True)

    wait_scat((NCHUNK - 2) % NB)
    wait_scat((NCHUNK - 1) % NB)
    plsc.subcore_barrier()

    @pl.when(s < NS - 1)
    def _():
        pltpu.sync_copy(agg.at[pl.ds(rbase, RPT)],
                        out_hbm.at[c].at[pl.ds(rbase, RPT)])

    @pl.when(s == NS - 1)
    def _():
        pltpu.sync_copy(agg.at[pl.ds((NS - 1) * RPT, RLAST)],
                        out_hbm.at[c].at[pl.ds((NS - 1) * RPT, RLAST)])


@functools.lru_cache(maxsize=1)
def _sc_edges_fn():
    mesh = plsc.VectorSubcoreMesh(core_axis_name="c", subcore_axis_name="s",
                                  num_cores=NC, num_subcores=NS)
    return pl.kernel(
        _sc_body,
        out_type=jax.ShapeDtypeStruct((NC, N, H), _f32),
        mesh=mesh,
        scratch_types=[
            pltpu.VMEM((NB, C), jnp.int32),
            pltpu.VMEM((NB, C), jnp.int32),
            pltpu.VMEM((NB, C // 2, H), jnp.int32),
            pltpu.VMEM((NB, C, H), _f32),
            pltpu.VMEM_SHARED((N, H), _f32),
        ] + [pltpu.SemaphoreType.DMA] * (5 * NB),
    )


def _sc_edges(hw, ew, src, dst, zrs):
    ew3 = ew.reshape(E // C, C // 2, H)
    return _sc_edges_fn()(hw, ew3, src, dst, zrs)


def _head_body(hs_ref, p_ref, batch_ref, gf_ref, wp_ref, bp_ref, wv_ref,
               bv_ref, logits_ref, value_ref):
    h = jnp.maximum(hs_ref[...] + p_ref[0] + p_ref[1], 0.0)
    seg = lax.broadcasted_iota(jnp.int32, (B, N), 0)
    onehot = jnp.where(seg == batch_ref[...], 1.0, 0.0).astype(_f32)
    sums = jnp.dot(onehot, h, preferred_element_type=_f32)
    counts = jnp.maximum(jnp.sum(onehot, axis=1, keepdims=True), 1.0)
    pooled = sums / counts
    feats = jnp.concatenate([pooled, gf_ref[...]], axis=1)
    logits_ref[...] = jnp.dot(feats, wp_ref[...], preferred_element_type=_f32) + bp_ref[...]
    value_ref[...] = jnp.dot(feats, wv_ref[...], preferred_element_type=_f32) + bv_ref[...]


def _head(hs, p, batch, gf, wp, bp, wv, bv):
    return pl.pallas_call(
        _head_body,
        out_shape=[
            jax.ShapeDtypeStruct((B, A), _f32),
            jax.ShapeDtypeStruct((B, 1), _f32),
        ],
    )(hs, p, batch.reshape(1, N), gf, wp, bp.reshape(1, A), wv,
      bv.reshape(1, 1))


def kernel(x, edge_index, edge_attr, batch, global_feats,
           W_self_0, W_msg_0, W_edge_0, b_0,
           W_self_1, W_msg_1, W_edge_1, b_1,
           W_self_2, W_msg_2, W_edge_2, b_2,
           W_pol, b_pol, W_val, b_val):
    src = edge_index[0]
    dst = edge_index[1]
    zrs = jnp.zeros((RPT, H), _f32)

    hw, hs = _mm1(x, W_msg_0, W_self_0, b_0)
    ew = _mme(edge_attr, W_edge_0)
    p = _sc_edges(hw, ew, src, dst, zrs)

    hw, hs = _mm2(hs, p, W_msg_1, W_self_1, b_1)
    ew = _mme(edge_attr, W_edge_1)
    p = _sc_edges(hw, ew, src, dst, zrs)

    hw, hs = _mm2(hs, p, W_msg_2, W_self_2, b_2)
    ew = _mme(edge_attr, W_edge_2)
    p = _sc_edges(hw, ew, src, dst, zrs)

    logits, value = _head(hs, p, batch, global_feats, W_pol, b_pol,
                          W_val, b_val)
    return logits, value


# speedup vs baseline: 1.4186x; 1.4186x over previous
import functools
import jax
import jax.numpy as jnp
from jax import lax
from jax.experimental import pallas as pl
from jax.experimental.pallas import tpu as pltpu
from jax.experimental.pallas import tpu_sc as plsc

N = 10000
E = 320000
D = 128
H = 128
DE = 16
GD = 64
A = 64
B = 8

NC = 2
NS = 16
NW = NC * NS
EW = E // NW
C = 40
NCHUNK = EW // C
NB = 5
NLOOP = NCHUNK // NB
RPT = 640
RLAST = N - (NS - 1) * RPT

_f32 = jnp.float32


def _mm1_body(x_ref, wm_ref, ws_ref, b_ref, hw_ref, hs_ref):
    xb = x_ref[...]
    hw_ref[...] = jnp.dot(xb, wm_ref[...], preferred_element_type=_f32)
    hs_ref[...] = jnp.dot(xb, ws_ref[...], preferred_element_type=_f32) + b_ref[...]


def _mm1(x, wm, ws, b, bn=2000):
    din = x.shape[1]
    grid = (N // bn,)
    return pl.pallas_call(
        _mm1_body,
        grid=grid,
        in_specs=[
            pl.BlockSpec((bn, din), lambda i: (i, 0)),
            pl.BlockSpec((din, H), lambda i: (0, 0)),
            pl.BlockSpec((din, H), lambda i: (0, 0)),
            pl.BlockSpec((1, H), lambda i: (0, 0)),
        ],
        out_specs=[
            pl.BlockSpec((bn, H), lambda i: (i, 0)),
            pl.BlockSpec((bn, H), lambda i: (i, 0)),
        ],
        out_shape=[
            jax.ShapeDtypeStruct((N, H), _f32),
            jax.ShapeDtypeStruct((N, H), _f32),
        ],
    )(x, wm, ws, b.reshape(1, H))


def _mm2_body(hs_ref, p_ref, wm_ref, ws_ref, b_ref, hw_ref, hs_o_ref):
    h = jnp.maximum(hs_ref[...] + p_ref[0] + p_ref[1], 0.0)
    hw_ref[...] = jnp.dot(h, wm_ref[...], preferred_element_type=_f32)
    hs_o_ref[...] = jnp.dot(h, ws_ref[...], preferred_element_type=_f32) + b_ref[...]


def _mm2(hs, p, wm, ws, b, bn=2000):
    grid = (N // bn,)
    return pl.pallas_call(
        _mm2_body,
        grid=grid,
        in_specs=[
            pl.BlockSpec((bn, H), lambda i: (i, 0)),
            pl.BlockSpec((2, bn, H), lambda i: (0, i, 0)),
            pl.BlockSpec((H, H), lambda i: (0, 0)),
            pl.BlockSpec((H, H), lambda i: (0, 0)),
            pl.BlockSpec((1, H), lambda i: (0, 0)),
        ],
        out_specs=[
            pl.BlockSpec((bn, H), lambda i: (i, 0)),
            pl.BlockSpec((bn, H), lambda i: (i, 0)),
        ],
        out_shape=[
            jax.ShapeDtypeStruct((N, H), _f32),
            jax.ShapeDtypeStruct((N, H), _f32),
        ],
    )(hs, p, wm, ws, b.reshape(1, H))


def _mme_body(ea_ref, we_ref, o_ref):
    o_ref[...] = jnp.dot(ea_ref[...], we_ref[...], preferred_element_type=_f32)


def _mme(edge_attr, we, be=8000):
    grid = (E // be,)
    return pl.pallas_call(
        _mme_body,
        grid=grid,
        in_specs=[
            pl.BlockSpec((be, DE), lambda i: (i, 0)),
            pl.BlockSpec((DE, H), lambda i: (0, 0)),
        ],
        out_specs=pl.BlockSpec((be, H), lambda i: (i, 0)),
        out_shape=jax.ShapeDtypeStruct((E, H), _f32),
    )(edge_attr, we)


def _sc_body(hw_hbm, ew_hbm, sd_hbm, zrs_hbm, out_hbm,
             sdix, ewb, agg, *sems):
    sem_sd = sems[0 * NB:1 * NB]
    sem_ew = sems[1 * NB:2 * NB]
    sem_hw = sems[2 * NB:3 * NB]
    sem_sc = sems[3 * NB:4 * NB]
    c = lax.axis_index("c")
    s = lax.axis_index("s")
    wid = s * NC + c
    ebase = wid * EW

    rbase = pl.multiple_of(s * RPT, 8)

    @pl.when(s < NS - 1)
    def _():
        pltpu.sync_copy(zrs_hbm, agg.at[pl.ds(rbase, RPT)])

    @pl.when(s == NS - 1)
    def _():
        pltpu.sync_copy(zrs_hbm.at[pl.ds(0, RLAST)],
                        agg.at[pl.ds((NS - 1) * RPT, RLAST)])

    plsc.subcore_barrier()

    def load(j, b):
        off = pl.multiple_of(ebase + j * C, 8)
        pltpu.async_copy(sd_hbm.at[wid * NCHUNK + j], sdix.at[b], sem_sd[b])
        pltpu.async_copy(ew_hbm.at[pl.ds(off, C)], ewb.at[b], sem_ew[b])

    def gath(b):
        pltpu.make_async_copy(sd_hbm.at[0], sdix.at[b], sem_sd[b]).wait()
        pltpu.make_async_copy(ew_hbm.at[pl.ds(0, C)], ewb.at[b],
                              sem_ew[b]).wait()
        pltpu.async_copy(hw_hbm.at[sdix.at[b, 0]], ewb.at[b], sem_hw[b],
                         add=True)

    def scat(b):
        pltpu.make_async_copy(hw_hbm.at[sdix.at[b, 0]], ewb.at[b],
                              sem_hw[b]).wait()

        def row(r, rc):
            for g in range(H // 16):
                sl = pl.ds(g * 16, 16)
                ewb[b, r, sl] = jnp.maximum(ewb[b, r, sl], 0.0)
            return rc

        lax.fori_loop(0, C, row, 0)
        pltpu.async_copy(ewb.at[b], agg.at[sdix.at[b, 1]], sem_sc[b],
                         add=True)

    def wait_scat(b):
        pltpu.make_async_copy(ewb.at[b], agg.at[sdix.at[b, 1]],
                              sem_sc[b]).wait()

    def body(t, first, last):
        for b in range(NB):
            j = t * NB + b
            bl = (b + 3) % NB
            if first and b < 2:
                pass
            else:
                wait_scat(bl)
            if not (last and b >= 2):
                load(j + 3, bl)
            if not (last and b == NB - 1):
                gath((b + 1) % NB)
            scat(b)

    for j in range(3):
        load(j, j)
    gath(0)

    body(0, True, False)
    lax.fori_loop(1, NLOOP - 1, lambda t, cy: (body(t, False, False), cy)[1],
                  0)
    body(NLOOP - 1, False, True)

    wait_scat((NCHUNK - 2) % NB)
    wait_scat((NCHUNK - 1) % NB)
    plsc.subcore_barrier()

    @pl.when(s < NS - 1)
    def _():
        pltpu.sync_copy(agg.at[pl.ds(rbase, RPT)],
                        out_hbm.at[c].at[pl.ds(rbase, RPT)])

    @pl.when(s == NS - 1)
    def _():
        pltpu.sync_copy(agg.at[pl.ds((NS - 1) * RPT, RLAST)],
                        out_hbm.at[c].at[pl.ds((NS - 1) * RPT, RLAST)])


@functools.lru_cache(maxsize=1)
def _sc_edges_fn():
    mesh = plsc.VectorSubcoreMesh(core_axis_name="c", subcore_axis_name="s",
                                  num_cores=NC, num_subcores=NS)
    return pl.kernel(
        _sc_body,
        out_type=jax.ShapeDtypeStruct((NC, N, H), _f32),
        mesh=mesh,
        scratch_types=[
            pltpu.VMEM((NB, 2, C), jnp.int32),
            pltpu.VMEM((NB, C, H), _f32),
            pltpu.VMEM_SHARED((N, H), _f32),
        ] + [pltpu.SemaphoreType.DMA] * (4 * NB),
    )


def _sc_edges(hw, ew, sd, zrs):
    return _sc_edges_fn()(hw, ew, sd, zrs)


def _head_body(hs_ref, p_ref, batch_ref, gf_ref, wp_ref, bp_ref, wv_ref,
               bv_ref, logits_ref, value_ref):
    h = jnp.maximum(hs_ref[...] + p_ref[0] + p_ref[1], 0.0)
    seg = lax.broadcasted_iota(jnp.int32, (B, N), 0)
    onehot = jnp.where(seg == batch_ref[...], 1.0, 0.0).astype(_f32)
    sums = jnp.dot(onehot, h, preferred_element_type=_f32)
    counts = jnp.maximum(jnp.sum(onehot, axis=1, keepdims=True), 1.0)
    pooled = sums / counts
    feats = jnp.concatenate([pooled, gf_ref[...]], axis=1)
    logits_ref[...] = jnp.dot(feats, wp_ref[...], preferred_element_type=_f32) + bp_ref[...]
    value_ref[...] = jnp.dot(feats, wv_ref[...], preferred_element_type=_f32) + bv_ref[...]


def _head(hs, p, batch, gf, wp, bp, wv, bv):
    return pl.pallas_call(
        _head_body,
        out_shape=[
            jax.ShapeDtypeStruct((B, A), _f32),
            jax.ShapeDtypeStruct((B, 1), _f32),
        ],
    )(hs, p, batch.reshape(1, N), gf, wp, bp.reshape(1, A), wv,
      bv.reshape(1, 1))


def kernel(x, edge_index, edge_attr, batch, global_feats,
           W_self_0, W_msg_0, W_edge_0, b_0,
           W_self_1, W_msg_1, W_edge_1, b_1,
           W_self_2, W_msg_2, W_edge_2, b_2,
           W_pol, b_pol, W_val, b_val):
    sd = (edge_index.reshape(2, NW, NCHUNK, C)
          .transpose(1, 2, 0, 3).reshape(NW * NCHUNK, 2, C))
    zrs = jnp.zeros((RPT, H), _f32)

    hw, hs = _mm1(x, W_msg_0, W_self_0, b_0)
    ew = _mme(edge_attr, W_edge_0)
    p = _sc_edges(hw, ew, sd, zrs)

    hw, hs = _mm2(hs, p, W_msg_1, W_self_1, b_1)
    ew = _mme(edge_attr, W_edge_1)
    p = _sc_edges(hw, ew, sd, zrs)

    hw, hs = _mm2(hs, p, W_msg_2, W_self_2, b_2)
    ew = _mme(edge_attr, W_edge_2)
    p = _sc_edges(hw, ew, sd, zrs)

    logits, value = _head(hs, p, batch, global_feats, W_pol, b_pol,
                          W_val, b_val)
    return logits, value


# speedup vs baseline: 1.4463x; 1.0195x over previous
import functools
import jax
import jax.numpy as jnp
from jax import lax
from jax.experimental import pallas as pl
from jax.experimental.pallas import tpu as pltpu
from jax.experimental.pallas import tpu_sc as plsc

N = 10000
E = 320000
D = 128
H = 128
DE = 16
GD = 64
A = 64
B = 8

NC = 2
NS = 16
NW = NC * NS
EW = E // NW
C = 40
NCHUNK = EW // C
NB = 5
NLOOP = NCHUNK // NB
RPT = 640
RLAST = N - (NS - 1) * RPT

_f32 = jnp.float32


def _mm1_body(x_ref, wm_ref, ws_ref, b_ref, hw_ref, hs_ref):
    xb = x_ref[...]
    hw_ref[...] = jnp.dot(xb, wm_ref[...], preferred_element_type=_f32)
    hs_ref[...] = jnp.dot(xb, ws_ref[...], preferred_element_type=_f32) + b_ref[...]


def _mm1(x, wm, ws, b, bn=2000):
    din = x.shape[1]
    grid = (N // bn,)
    return pl.pallas_call(
        _mm1_body,
        grid=grid,
        in_specs=[
            pl.BlockSpec((bn, din), lambda i: (i, 0)),
            pl.BlockSpec((din, H), lambda i: (0, 0)),
            pl.BlockSpec((din, H), lambda i: (0, 0)),
            pl.BlockSpec((1, H), lambda i: (0, 0)),
        ],
        out_specs=[
            pl.BlockSpec((bn, H), lambda i: (i, 0)),
            pl.BlockSpec((bn, H), lambda i: (i, 0)),
        ],
        out_shape=[
            jax.ShapeDtypeStruct((N, H), _f32),
            jax.ShapeDtypeStruct((N, H), _f32),
        ],
    )(x, wm, ws, b.reshape(1, H))


def _mm2_body(hs_ref, p_ref, wm_ref, ws_ref, b_ref, hw_ref, hs_o_ref):
    h = jnp.maximum(hs_ref[...] + p_ref[0] + p_ref[1], 0.0)
    hw_ref[...] = jnp.dot(h, wm_ref[...], preferred_element_type=_f32)
    hs_o_ref[...] = jnp.dot(h, ws_ref[...], preferred_element_type=_f32) + b_ref[...]


def _mm2(hs, p, wm, ws, b, bn=2000):
    grid = (N // bn,)
    return pl.pallas_call(
        _mm2_body,
        grid=grid,
        in_specs=[
            pl.BlockSpec((bn, H), lambda i: (i, 0)),
            pl.BlockSpec((2, bn, H), lambda i: (0, i, 0)),
            pl.BlockSpec((H, H), lambda i: (0, 0)),
            pl.BlockSpec((H, H), lambda i: (0, 0)),
            pl.BlockSpec((1, H), lambda i: (0, 0)),
        ],
        out_specs=[
            pl.BlockSpec((bn, H), lambda i: (i, 0)),
            pl.BlockSpec((bn, H), lambda i: (i, 0)),
        ],
        out_shape=[
            jax.ShapeDtypeStruct((N, H), _f32),
            jax.ShapeDtypeStruct((N, H), _f32),
        ],
    )(hs, p, wm, ws, b.reshape(1, H))


def _mme_body(ea_ref, we_ref, o_ref):
    o_ref[...] = jnp.dot(ea_ref[...], we_ref[...], preferred_element_type=_f32)


def _mme(edge_attr, we, be=8000):
    grid = (E // be,)
    return pl.pallas_call(
        _mme_body,
        grid=grid,
        in_specs=[
            pl.BlockSpec((be, DE), lambda i: (i, 0)),
            pl.BlockSpec((DE, H), lambda i: (0, 0)),
        ],
        out_specs=pl.BlockSpec((be, H), lambda i: (i, 0)),
        out_shape=jax.ShapeDtypeStruct((E, H), _f32),
    )(edge_attr, we)


def _sc_body(hw_hbm, ew_hbm, src_hbm, dst_hbm, zrs_hbm, out_hbm,
             sidx, didx, ewb, agg, *sems):
    sem_si = sems[0 * NB:1 * NB]
    sem_di = sems[1 * NB:2 * NB]
    sem_ew = sems[2 * NB:3 * NB]
    sem_hw = sems[3 * NB:4 * NB]
    sem_sc = sems[4 * NB:5 * NB]
    c = lax.axis_index("c")
    s = lax.axis_index("s")
    wid = s * NC + c
    ebase = wid * EW

    rbase = pl.multiple_of(s * RPT, 8)

    @pl.when(s < NS - 1)
    def _():
        pltpu.sync_copy(zrs_hbm, agg.at[pl.ds(rbase, RPT)])

    @pl.when(s == NS - 1)
    def _():
        pltpu.sync_copy(zrs_hbm.at[pl.ds(0, RLAST)],
                        agg.at[pl.ds((NS - 1) * RPT, RLAST)])

    plsc.subcore_barrier()

    def load(j, b):
        off = pl.multiple_of(ebase + j * C, 8)
        pltpu.async_copy(src_hbm.at[pl.ds(off, C)], sidx.at[b], sem_si[b])
        pltpu.async_copy(dst_hbm.at[pl.ds(off, C)], didx.at[b], sem_di[b])
        pltpu.async_copy(ew_hbm.at[pl.ds(off, C)], ewb.at[b], sem_ew[b])

    def gath(b):
        pltpu.make_async_copy(src_hbm.at[pl.ds(0, C)], sidx.at[b],
                              sem_si[b]).wait()
        pltpu.make_async_copy(ew_hbm.at[pl.ds(0, C)], ewb.at[b],
                              sem_ew[b]).wait()
        pltpu.async_copy(hw_hbm.at[sidx.at[b]], ewb.at[b], sem_hw[b],
                         add=True)

    def scat(b):
        pltpu.make_async_copy(hw_hbm.at[sidx.at[b]], ewb.at[b],
                              sem_hw[b]).wait()
        pltpu.make_async_copy(dst_hbm.at[pl.ds(0, C)], didx.at[b],
                              sem_di[b]).wait()

        def row(r, rc):
            for g in range(H // 16):
                sl = pl.ds(g * 16, 16)
                ewb[b, r, sl] = jnp.maximum(ewb[b, r, sl], 0.0)
            return rc

        lax.fori_loop(0, C, row, 0)
        pltpu.async_copy(ewb.at[b], agg.at[didx.at[b]], sem_sc[b], add=True)

    def wait_scat(b):
        pltpu.make_async_copy(ewb.at[b], agg.at[didx.at[b]],
                              sem_sc[b]).wait()

    def body(t, first, last):
        for b in range(NB):
            j = t * NB + b
            bl = (b + 3) % NB
            if first and b < 2:
                pass
            else:
                wait_scat(bl)
            if not (last and b >= 2):
                load(j + 3, bl)
            if not (last and b == NB - 1):
                gath((b + 1) % NB)
            scat(b)

    for j in range(3):
        load(j, j)
    gath(0)

    body(0, True, False)
    lax.fori_loop(1, NLOOP - 1, lambda t, cy: (body(t, False, False), cy)[1],
                  0)
    body(NLOOP - 1, False, True)

    wait_scat((NCHUNK - 2) % NB)
    wait_scat((NCHUNK - 1) % NB)
    plsc.subcore_barrier()

    @pl.when(s < NS - 1)
    def _():
        pltpu.sync_copy(agg.at[pl.ds(rbase, RPT)],
                        out_hbm.at[c].at[pl.ds(rbase, RPT)])

    @pl.when(s == NS - 1)
    def _():
        pltpu.sync_copy(agg.at[pl.ds((NS - 1) * RPT, RLAST)],
                        out_hbm.at[c].at[pl.ds((NS - 1) * RPT, RLAST)])


@functools.lru_cache(maxsize=1)
def _sc_edges_fn():
    mesh = plsc.VectorSubcoreMesh(core_axis_name="c", subcore_axis_name="s",
                                  num_cores=NC, num_subcores=NS)
    return pl.kernel(
        _sc_body,
        out_type=jax.ShapeDtypeStruct((NC, N, H), _f32),
        mesh=mesh,
        scratch_types=[
            pltpu.VMEM((NB, C), jnp.int32),
            pltpu.VMEM((NB, C), jnp.int32),
            pltpu.VMEM((NB, C, H), _f32),
            pltpu.VMEM_SHARED((N, H), _f32),
        ] + [pltpu.SemaphoreType.DMA] * (5 * NB),
    )


def _sc_edges(hw, ew, src, dst, zrs):
    return _sc_edges_fn()(hw, ew, src, dst, zrs)


def _head_body(hs_ref, p_ref, batch_ref, gf_ref, wp_ref, bp_ref, wv_ref,
               bv_ref, logits_ref, value_ref):
    h = jnp.maximum(hs_ref[...] + p_ref[0] + p_ref[1], 0.0)
    seg = lax.broadcasted_iota(jnp.int32, (B, N), 0)
    onehot = jnp.where(seg == batch_ref[...], 1.0, 0.0).astype(_f32)
    sums = jnp.dot(onehot, h, preferred_element_type=_f32)
    counts = jnp.maximum(jnp.sum(onehot, axis=1, keepdims=True), 1.0)
    pooled = sums / counts
    feats = jnp.concatenate([pooled, gf_ref[...]], axis=1)
    logits_ref[...] = jnp.dot(feats, wp_ref[...], preferred_element_type=_f32) + bp_ref[...]
    value_ref[...] = jnp.dot(feats, wv_ref[...], preferred_element_type=_f32) + bv_ref[...]


def _head(hs, p, batch, gf, wp, bp, wv, bv):
    return pl.pallas_call(
        _head_body,
        out_shape=[
            jax.ShapeDtypeStruct((B, A), _f32),
            jax.ShapeDtypeStruct((B, 1), _f32),
        ],
    )(hs, p, batch.reshape(1, N), gf, wp, bp.reshape(1, A), wv,
      bv.reshape(1, 1))


def kernel(x, edge_index, edge_attr, batch, global_feats,
           W_self_0, W_msg_0, W_edge_0, b_0,
           W_self_1, W_msg_1, W_edge_1, b_1,
           W_self_2, W_msg_2, W_edge_2, b_2,
           W_pol, b_pol, W_val, b_val):
    src = edge_index[0]
    dst = edge_index[1]
    zrs = jnp.zeros((RPT, H), _f32)

    hw, hs = _mm1(x, W_msg_0, W_self_0, b_0)
    ew = _mme(edge_attr, W_edge_0)
    p = _sc_edges(hw, ew, src, dst, zrs)

    hw, hs = _mm2(hs, p, W_msg_1, W_self_1, b_1)
    ew = _mme(edge_attr, W_edge_1)
    p = _sc_edges(hw, ew, src, dst, zrs)

    hw, hs = _mm2(hs, p, W_msg_2, W_self_2, b_2)
    ew = _mme(edge_attr, W_edge_2)
    p = _sc_edges(hw, ew, src, dst, zrs)

    logits, value = _head(hs, p, batch, global_feats, W_pol, b_pol,
                          W_val, b_val)
    return logits, value
